# baseline (speedup 1.0000x reference)
import functools

import jax
import jax.numpy as jnp
from jax import lax
from jax.experimental import pallas as pl
from jax.experimental.pallas import tpu as pltpu
from jax.experimental.pallas import tpu_sc as plsc

N = 10000
E = 320000
PE = 100000
D = 128

NC = 2
NS = 16
NW = NC * NS

C = 128
SB = 8
DB = 8
EP = 327680
PEP = 114688
EP2 = 2 * PEP
NP = 10240
RPW = NP // NS

_mesh = lambda: plsc.VectorSubcoreMesh(core_axis_name="c", subcore_axis_name="s", num_cores=NC, num_subcores=NS)


def _make_spmm(with_deg):
  out_type = [jax.ShapeDtypeStruct((NC * NP, D), jnp.float32)]
  if with_deg:
    out_type.append(jax.ShapeDtypeStruct((NW * NP,), jnp.float32))
  scratch = [
      pltpu.VMEM((SB, C), jnp.int32),
      pltpu.VMEM((SB, C), jnp.int32),
      pltpu.VMEM((C,), jnp.int32),
      pltpu.VMEM((C,), jnp.int32),
      pltpu.VMEM((C,), jnp.int32),
      pltpu.VMEM((C,), jnp.int32),
      pltpu.VMEM((C, D), jnp.float32),
      pltpu.VMEM((C, D), jnp.float32),
      pltpu.VMEM_SHARED((NP, D), jnp.float32),
      pltpu.SemaphoreType.DMA,
      pltpu.SemaphoreType.DMA,
  ]
  if with_deg:
    scratch += [
        pltpu.VMEM((NP,), jnp.float32),
    ]

  @functools.partial(pl.kernel, out_type=out_type, mesh=_mesh(),
                     scratch_types=scratch,
                     compiler_params=pltpu.CompilerParams(
                         needs_layout_passes=False))
  def spmm(*refs):
    if with_deg:
      (table_h, src_h, dst_h, out_h, deg_h,
       srcbt_v, dstbt_v, sca_v, scb_v, dca_v, dcb_v, rows_v, rowsb_v,
       acc_sh, sem, semb, hist_v) = refs
    else:
      (table_h, src_h, dst_h, out_h,
       srcbt_v, dstbt_v, sca_v, scb_v, dca_v, dcb_v, rows_v, rowsb_v,
       acc_sh, sem, semb) = refs
    cid = lax.axis_index("c")
    sid = lax.axis_index("s")
    wid = sid * NC + cid

    zero16 = jnp.zeros((16,), jnp.float32)
    def zrow(i, _):
      for j in range(D // 16):
        rows_v[i, pl.ds(16 * j, 16)] = zero16
      return 0
    lax.fori_loop(0, C, zrow, 0)
    if with_deg:
      def zhist(i, _):
        hist_v[pl.ds(i * 16, 16)] = zero16
        return 0
      lax.fori_loop(0, NP // 16, zhist, 0)

    row0 = sid * RPW
    def zacc(m, _):
      pltpu.sync_copy(rows_v, acc_sh.at[pl.ds(row0 + m * C, C)])
      return 0
    lax.fori_loop(0, RPW // C, zacc, 0)

    plsc.subcore_barrier()

    nch = EP // (NW * C)
    nbat = nch // SB
    base0 = wid * nch * C
    row_b0 = wid * nch

    def hist_update(slicer):
      if with_deg:
        for q in range(C // 16):
          idx16 = slicer(q)
          cnt, last = plsc.scan_count(idx16)
          plsc.addupdate_scatter(hist_v, [idx16], cnt.astype(jnp.float32),
                                 mask=last)

    bufs = (rows_v, rowsb_v)
    sems = (sem, semb)
    scur = (sca_v, scb_v)
    dcur = (dca_v, dcb_v)

    def stage(j):
      p = j % 2
      for q in range(C // 16):
        scur[p][pl.ds(q * 16, 16)] = srcbt_v[j, pl.ds(q * 16, 16)]
        dcur[p][pl.ds(q * 16, 16)] = dstbt_v[j, pl.ds(q * 16, 16)]
      pltpu.async_copy(table_h.at[scur[p]], bufs[p], sems[p])

    def consume(j):
      p = j % 2
      hist_update(lambda q, pp=p: dcur[pp][pl.ds(q * 16, 16)])
      pltpu.make_async_copy(table_h.at[pl.ds(0, C)], bufs[p], sems[p]).wait()
      pltpu.sync_copy(bufs[p], acc_sh.at[dcur[p]], add=True)

    def load_batch(bt):
      pltpu.sync_copy(src_h.at[pl.ds(row_b0 + bt * SB, SB)], srcbt_v)
      pltpu.sync_copy(dst_h.at[pl.ds(row_b0 + bt * SB, SB)], dstbt_v)

    load_batch(0)
    stage(0)

    def step_bt(bt, _):
      for j in range(1, SB):
        stage(j)
        consume(j - 1)
      @pl.when(bt < nbat - 1)
      def _():
        load_batch(bt + 1)
        stage(0)
        consume(SB - 1)
      return 0
    lax.fori_loop(0, nbat, step_bt, 0)
    consume(SB - 1)

    plsc.subcore_barrier()

    def cp(m, _):
      r = row0 + m * C
      pltpu.sync_copy(acc_sh.at[pl.ds(r, C)], out_h.at[pl.ds(cid * NP + r, C)])
      return 0
    lax.fori_loop(0, RPW // C, cp, 0)

    if with_deg:
      pltpu.sync_copy(hist_v, deg_h.at[pl.ds(wid * NP, NP)])

  return spmm


_spmm_deg = _make_spmm(True)
_spmm = _make_spmm(False)


def _tc_layer1(parts, deg, x, wlT, bl, wrT):
  R = 400
  def body(parts_ref, deg_ref, x_ref, wlT_ref, bl_ref, wrT_ref, out_ref):
    aggsum = parts_ref[0] + parts_ref[1]
    d = jnp.sum(deg_ref[...], axis=0)
    recip = 1.0 / jnp.maximum(d, 1.0)
    y = (jnp.dot(aggsum, wlT_ref[...], preferred_element_type=jnp.float32)
         * recip + bl_ref[...]
         + jnp.dot(x_ref[...], wrT_ref[...], preferred_element_type=jnp.float32))
    out_ref[...] = jnp.maximum(y, 0.0)
  return pl.pallas_call(
      body,
      grid=(N // R,),
      in_specs=[
          pl.BlockSpec((2, R, D), lambda i: (0, i, 0)),
          pl.BlockSpec((NW, R, 1), lambda i: (0, i, 0)),
          pl.BlockSpec((R, D), lambda i: (i, 0)),
          pl.BlockSpec((D, D), lambda i: (0, 0)),
          pl.BlockSpec((1, D), lambda i: (0, 0)),
          pl.BlockSpec((D, D), lambda i: (0, 0)),
      ],
      out_specs=pl.BlockSpec((R, D), lambda i: (i, 0)),
      out_shape=jax.ShapeDtypeStruct((N, D), jnp.float32),
  )(parts, deg, x, wlT, bl, wrT)


def _tc_layer2(parts, deg, z1, wlT, bl, wrT, walT, ba, warT):
  R = 400
  def body(parts_ref, deg_ref, z1_ref, wlT_ref, bl_ref, wrT_ref,
           walT_ref, ba_ref, warT_ref, u_ref, v_ref):
    aggsum = parts_ref[0] + parts_ref[1]
    d = jnp.sum(deg_ref[...], axis=0)
    recip = 1.0 / jnp.maximum(d, 1.0)
    z2 = (jnp.dot(aggsum, wlT_ref[...], preferred_element_type=jnp.float32)
          * recip + bl_ref[...]
          + jnp.dot(z1_ref[...], wrT_ref[...], preferred_element_type=jnp.float32))
    u_ref[...] = jnp.dot(z2, walT_ref[...],
                         preferred_element_type=jnp.float32) + ba_ref[...]
    v_ref[...] = jnp.dot(z2, warT_ref[...],
                         preferred_element_type=jnp.float32)
  return pl.pallas_call(
      body,
      grid=(N // R,),
      in_specs=[
          pl.BlockSpec((2, R, D), lambda i: (0, i, 0)),
          pl.BlockSpec((NW, R, 1), lambda i: (0, i, 0)),
          pl.BlockSpec((R, D), lambda i: (i, 0)),
          pl.BlockSpec((D, D), lambda i: (0, 0)),
          pl.BlockSpec((1, D), lambda i: (0, 0)),
          pl.BlockSpec((D, D), lambda i: (0, 0)),
          pl.BlockSpec((D, D), lambda i: (0, 0)),
          pl.BlockSpec((1, D), lambda i: (0, 0)),
          pl.BlockSpec((D, D), lambda i: (0, 0)),
      ],
      out_specs=[pl.BlockSpec((R, D), lambda i: (i, 0)),
                 pl.BlockSpec((R, D), lambda i: (i, 0))],
      out_shape=[jax.ShapeDtypeStruct((N, D), jnp.float32),
                 jax.ShapeDtypeStruct((N, D), jnp.float32)],
  )(parts, deg, z1, wlT, bl, wrT, walT, ba, warT)


@functools.partial(
    pl.kernel,
    out_type=jax.ShapeDtypeStruct((EP2,), jnp.float32),
    mesh=_mesh(),
    scratch_types=[
        pltpu.VMEM((DB, C), jnp.int32),
        pltpu.VMEM((DB, C), jnp.int32),
        pltpu.VMEM((C,), jnp.int32),
        pltpu.VMEM((C,), jnp.int32),
        pltpu.VMEM((C,), jnp.int32),
        pltpu.VMEM((C,), jnp.int32),
        pltpu.VMEM((C, D), jnp.float32),
        pltpu.VMEM((C, D), jnp.float32),
        pltpu.VMEM((C, D), jnp.float32),
        pltpu.VMEM((C, D), jnp.float32),
        pltpu.VMEM((C,), jnp.float32),
        pltpu.VMEM((D + 16,), jnp.float32),
        pltpu.SemaphoreType.DMA,
        pltpu.SemaphoreType.DMA,
    ],
    compiler_params=pltpu.CompilerParams(needs_layout_passes=False),
)
def _decode(u_h, v_h, src_h, dst_h, wbb_h, out_h,
            srcbt_v, dstbt_v, sca_v, scb_v, dca_v, dcb_v,
            u_rows, v_rows, u_rowsb, v_rowsb,
            pred_v, wbb_v, sem_a, sem_b):
  cid = lax.axis_index("c")
  sid = lax.axis_index("s")
  wid = sid * NC + cid
  pltpu.sync_copy(wbb_h, wbb_v)
  wbs = [wbb_v[pl.ds(16 * j, 16)] for j in range(D // 16)]
  bbv = wbb_v[pl.ds(D, 16)]
  lanes = lax.iota(jnp.int32, 16)

  ubufs = (u_rows, u_rowsb)
  vbufs = (v_rows, v_rowsb)
  sems = (sem_a, sem_b)

  def compute(ur, vr):
    def group(g, _):
      out_vec = jnp.zeros((16,), jnp.float32)
      for i in range(16):
        e = g * 16 + i
        acc = bbv
        for j in range(D // 16):
          t = jnp.maximum(ur[e, pl.ds(16 * j, 16)]
                          + vr[e, pl.ds(16 * j, 16)], 0.0)
          acc = acc + t * wbs[j]
        out_vec = jnp.where(lanes == i, jnp.sum(acc), out_vec)
      pred_v[pl.ds(g * 16, 16)] = out_vec
      return 0
    lax.fori_loop(0, C // 16, group, 0)

  nch = EP2 // (NW * C)
  nbat = nch // DB
  base0 = wid * nch * C
  row_b0 = wid * nch

  scur = (sca_v, scb_v)
  dcur = (dca_v, dcb_v)

  def stage(j, p):
    for q in range(C // 16):
      scur[p][pl.ds(q * 16, 16)] = srcbt_v[j, pl.ds(q * 16, 16)]
      dcur[p][pl.ds(q * 16, 16)] = dstbt_v[j, pl.ds(q * 16, 16)]
    pltpu.async_copy(u_h.at[scur[p]], ubufs[p], sems[p])
    pltpu.async_copy(v_h.at[dcur[p]], vbufs[p], sems[p])

  def consume_p(p, b_out):
    s = sems[p]
    pltpu.make_async_copy(u_h.at[pl.ds(0, C)], ubufs[p], s).wait()
    pltpu.make_async_copy(u_h.at[pl.ds(0, C)], vbufs[p], s).wait()
    compute(ubufs[p], vbufs[p])
    pltpu.sync_copy(pred_v, out_h.at[pl.ds(b_out, C)])

  def load_batch(bt):
    pltpu.sync_copy(src_h.at[pl.ds(row_b0 + bt * DB, DB)], srcbt_v)
    pltpu.sync_copy(dst_h.at[pl.ds(row_b0 + bt * DB, DB)], dstbt_v)

  load_batch(0)
  stage(0, 0)

  def step_bt(bt, _):
    b = base0 + bt * DB * C
    def pair(m, _):
      j1 = 2 * m + 1
      stage(j1, 1)
      consume_p(0, b + (j1 - 1) * C)
      stage(j1 + 1, 0)
      consume_p(1, b + j1 * C)
      return 0
    lax.fori_loop(0, (DB - 2) // 2, pair, 0)
    stage(DB - 1, 1)
    consume_p(0, b + (DB - 2) * C)
    @pl.when(bt < nbat - 1)
    def _():
      load_batch(bt + 1)
      stage(0, 0)
      consume_p(1, b + (DB - 1) * C)
    return 0
  lax.fori_loop(0, nbat, step_bt, 0)
  consume_p(1, base0 + (nch - 1) * C)


def kernel(x, W1l, b1l, W1r, W2l, b2l, W2r, Wa, ba, Wb, bb,
           edge_index, pos_edge_index, neg_edge_index):
  i32 = jnp.int32
  f32 = jnp.float32
  src = edge_index[0].astype(i32)
  dst = edge_index[1].astype(i32)
  pad_e = EP - E
  src_p = jnp.concatenate([src, jnp.zeros((pad_e,), i32)])
  dst_p = jnp.concatenate([dst, jnp.full((pad_e,), N, i32)])

  src2 = src_p.reshape(EP // C, C)
  dst2 = dst_p.reshape(EP // C, C)
  parts1, degp = _spmm_deg(x, src2, dst2)
  parts1 = parts1.reshape(NC, NP, D)
  degp = degp.reshape(NW, NP, 1)
  z1 = _tc_layer1(parts1, degp, x, W1l.T, b1l.reshape(1, D), W1r.T)

  parts2, = _spmm(z1, src2, dst2)
  parts2 = parts2.reshape(NC, NP, D)
  U, V = _tc_layer2(parts2, degp, z1, W2l.T, b2l.reshape(1, D), W2r.T,
                    Wa[:, :D].T, ba.reshape(1, D), Wa[:, D:].T)

  padp = PEP - PE
  zi = jnp.zeros((padp,), i32)
  src_all = jnp.concatenate([pos_edge_index[0].astype(i32), zi,
                             neg_edge_index[0].astype(i32), zi])
  dst_all = jnp.concatenate([pos_edge_index[1].astype(i32), zi,
                             neg_edge_index[1].astype(i32), zi])
  wbb = jnp.concatenate([Wb[0], bb, jnp.zeros((15,), f32)])

  preds = _decode(U, V, src_all.reshape(EP2 // C, C),
                  dst_all.reshape(EP2 // C, C), wbb)
  return preds[:PE], preds[PEP:PEP + PE]
